# baseline (speedup 1.0000x reference)
import functools

import jax
import jax.numpy as jnp
from jax import lax
from jax.experimental import pallas as pl
from jax.experimental.pallas import tpu as pltpu
from jax.experimental.pallas import tpu_sc as plsc

VOCAB = 1000000
HIDDEN = 64

NC = 2
NS = 16
NW = NC * NS

B_TOTAL = 4096 * 200
B_PER_W = B_TOTAL // NW
IDX_MINOR = 128
CHUNK_K = 8
CHUNK = CHUNK_K * IDX_MINOR
N_CHUNKS = B_PER_W // CHUNK

_mesh = plsc.VectorSubcoreMesh(core_axis_name="c", subcore_axis_name="s")


@functools.partial(
    pl.kernel,
    mesh=_mesh,
    compiler_params=pltpu.CompilerParams(use_tc_tiling_on_sc=False),
    out_type=jax.ShapeDtypeStruct((B_TOTAL, HIDDEN), jnp.float32),
    scratch_types=[
        pltpu.VMEM((CHUNK_K, IDX_MINOR), jnp.int32),
        pltpu.VMEM((CHUNK, HIDDEN), jnp.float32),
        pltpu.SemaphoreType.DMA,
    ],
)
def _embed(idx_hbm, table_hbm, out_hbm, idx_v, rows_v, sem):
    wid = lax.axis_index("s") * NC + lax.axis_index("c")
    base = wid * B_PER_W

    def body(i, _):
        off = pl.multiple_of(base + i * CHUNK, CHUNK)
        row = pl.multiple_of(base // IDX_MINOR + i * CHUNK_K, CHUNK_K)
        pltpu.sync_copy(idx_hbm.at[pl.ds(row, CHUNK_K)], idx_v)
        for j in range(CHUNK_K):
            pltpu.async_copy(
                table_hbm.at[idx_v.at[j]],
                rows_v.at[pl.ds(j * IDX_MINOR, IDX_MINOR)],
                sem,
            ).wait()
        pltpu.sync_copy(rows_v, out_hbm.at[pl.ds(off, CHUNK)])
        return ()

    lax.fori_loop(0, N_CHUNKS, body, ())


def kernel(input_ids, embed_tokens):
    idx2d = input_ids.reshape(B_TOTAL // IDX_MINOR, IDX_MINOR).astype(jnp.int32)
    out = _embed(idx2d, embed_tokens)
    return out.reshape(input_ids.shape + (HIDDEN,))


# speedup vs baseline: 1.1080x; 1.1080x over previous
import functools

import jax
import jax.numpy as jnp
from jax import lax
from jax.experimental import pallas as pl
from jax.experimental.pallas import tpu as pltpu
from jax.experimental.pallas import tpu_sc as plsc

VOCAB = 1000000
HIDDEN = 64

NC = 2
NS = 16
NW = NC * NS

B_TOTAL = 4096 * 200
B_PER_W = B_TOTAL // NW
CHUNK = 1024
N_CHUNKS = B_PER_W // CHUNK

_mesh = plsc.VectorSubcoreMesh(core_axis_name="c", subcore_axis_name="s")


@functools.partial(
    pl.kernel,
    mesh=_mesh,
    compiler_params=pltpu.CompilerParams(use_tc_tiling_on_sc=False),
    out_type=jax.ShapeDtypeStruct((B_TOTAL, HIDDEN), jnp.float32),
    scratch_types=[
        pltpu.VMEM((B_PER_W,), jnp.int32),
        pltpu.VMEM((CHUNK, HIDDEN), jnp.float32),
        pltpu.SemaphoreType.DMA,
    ],
)
def _embed(idx_hbm, table_hbm, out_hbm, idx_v, rows_v, sem):
    wid = lax.axis_index("s") * NC + lax.axis_index("c")
    base = pl.multiple_of(wid * B_PER_W, B_PER_W)
    pltpu.sync_copy(idx_hbm.at[pl.ds(base, B_PER_W)], idx_v)

    def body(i, _):
        off = pl.multiple_of(base + i * CHUNK, CHUNK)
        pltpu.async_copy(
            table_hbm.at[idx_v.at[pl.ds(i * CHUNK, CHUNK)]],
            rows_v,
            sem,
        ).wait()
        pltpu.sync_copy(rows_v, out_hbm.at[pl.ds(off, CHUNK)])
        return ()

    lax.fori_loop(0, N_CHUNKS, body, ())


def kernel(input_ids, embed_tokens):
    flat = input_ids.reshape(-1).astype(jnp.int32)
    out = _embed(flat, embed_tokens)
    return out.reshape(input_ids.shape + (HIDDEN,))


# speedup vs baseline: 1.1199x; 1.0107x over previous
import functools

import jax
import jax.numpy as jnp
from jax import lax
from jax.experimental import pallas as pl
from jax.experimental.pallas import tpu as pltpu
from jax.experimental.pallas import tpu_sc as plsc

VOCAB = 1000000
HIDDEN = 64

NC = 2
NS = 16
NW = NC * NS

B_TOTAL = 4096 * 200
B_PER_W = B_TOTAL // NW
CHUNK = 400
N_CHUNKS = B_PER_W // CHUNK
NBUF = 4
K = 3
NQ = N_CHUNKS // NBUF

_mesh = plsc.VectorSubcoreMesh(core_axis_name="c", subcore_axis_name="s")


@functools.partial(
    pl.kernel,
    mesh=_mesh,
    compiler_params=pltpu.CompilerParams(use_tc_tiling_on_sc=False),
    out_type=jax.ShapeDtypeStruct((B_TOTAL, HIDDEN), jnp.float32),
    scratch_types=[
        pltpu.VMEM((B_PER_W,), jnp.int32),
        pltpu.VMEM((NBUF, CHUNK, HIDDEN), jnp.float32),
        pltpu.SemaphoreType.DMA,
        pltpu.SemaphoreType.DMA,
    ],
)
def _embed(idx_hbm, table_hbm, out_hbm, idx_v, rows_v, gsem, ssem):
    wid = lax.axis_index("s") * NC + lax.axis_index("c")
    base = pl.multiple_of(wid * B_PER_W, B_PER_W)
    pltpu.sync_copy(idx_hbm.at[pl.ds(base, B_PER_W)], idx_v)

    def fire_gather(j, b):
        off = pl.multiple_of(j * CHUNK, 8)
        pltpu.async_copy(
            table_hbm.at[idx_v.at[pl.ds(off, CHUNK)]], rows_v.at[b], gsem
        )

    def wait_gather(b):
        pltpu.make_async_copy(
            table_hbm.at[idx_v.at[pl.ds(0, CHUNK)]], rows_v.at[b], gsem
        ).wait()

    def fire_store(j, b):
        off = pl.multiple_of(base + j * CHUNK, 8)
        pltpu.async_copy(rows_v.at[b], out_hbm.at[pl.ds(off, CHUNK)], ssem)

    def wait_store():
        pltpu.make_async_copy(
            rows_v.at[0], out_hbm.at[pl.ds(0, CHUNK)], ssem
        ).wait()

    for j in range(K):
        fire_gather(j, j)

    for b in range(NBUF):
        if b >= 1:
            wait_store()
        wait_gather(b)
        fire_store(b, b)
        fire_gather(b + K, (b + K) % NBUF)

    def round_(o, _):
        for b in range(NBUF):
            i = o * NBUF + b
            wait_store()
            wait_gather(b)
            fire_store(i, b)
            fire_gather(i + K, (b + K) % NBUF)
        return ()

    lax.fori_loop(1, NQ - 1, round_, ())

    for b in range(NBUF):
        i = (NQ - 1) * NBUF + b
        wait_store()
        wait_gather(b)
        fire_store(i, b)
        if i + K < N_CHUNKS:
            fire_gather(i + K, (b + K) % NBUF)
    wait_store()


def kernel(input_ids, embed_tokens):
    flat = input_ids.reshape(-1).astype(jnp.int32)
    out = _embed(flat, embed_tokens)
    return out.reshape(input_ids.shape + (HIDDEN,))
